# Initial kernel scaffold; baseline (speedup 1.0000x reference)
import functools

import jax
import jax.numpy as jnp
from jax import lax
from jax.experimental import pallas as pl
from jax.experimental.pallas import tpu as pltpu
from jax.experimental.pallas import tpu_sc as plsc

_NUM_STOCKS = 100000
_EMBED_DIM = 32
_BATCH = 16384
_SEQ_LEN = 50

_B = _BATCH * _SEQ_LEN
_NC = 2
_NS = 16
_NW = _NC * _NS
_B_PER_W = _B // _NW
_IDXW = 128
_K = 20
_CHUNK = _K * _IDXW
_N_CHUNKS = _B_PER_W // _CHUNK

_mesh = plsc.VectorSubcoreMesh(core_axis_name="c", subcore_axis_name="s")


@functools.partial(
    pl.kernel,
    mesh=_mesh,
    out_type=jax.ShapeDtypeStruct((_B, _EMBED_DIM), jnp.float32),
    scratch_types=[
        pltpu.VMEM((_K, _IDXW), jnp.int32),
        pltpu.VMEM((_CHUNK, _EMBED_DIM), jnp.float32),
        pltpu.SemaphoreType.DMA,
    ],
)
def _emb_lookup(idx_hbm, table_hbm, out_hbm, idx_v, rows_v, sem):
    wid = lax.axis_index("s") * _NC + lax.axis_index("c")
    base = wid * _B_PER_W

    def chunk_body(i, carry):
        off = base + i * _CHUNK
        pltpu.sync_copy(idx_hbm.at[pl.ds(off // _IDXW, _K)], idx_v)
        copies = []
        for j in range(_K):
            copies.append(
                pltpu.async_copy(
                    table_hbm.at[idx_v.at[j]],
                    rows_v.at[pl.ds(j * _IDXW, _IDXW)],
                    sem,
                )
            )
        for cp in copies:
            cp.wait()
        pltpu.sync_copy(rows_v, out_hbm.at[pl.ds(off, _CHUNK)])
        return carry

    lax.fori_loop(0, _N_CHUNKS, chunk_body, 0)


def kernel(stock_ids, weight):
    idx2d = stock_ids.reshape(_B // _IDXW, _IDXW)
    out = _emb_lookup(idx2d, weight)
    return out.reshape(_BATCH, _SEQ_LEN, _EMBED_DIM)


# speedup vs baseline: 2.9965x; 2.9965x over previous
import functools

import jax
import jax.numpy as jnp
from jax import lax
from jax.experimental import pallas as pl
from jax.experimental.pallas import tpu as pltpu
from jax.experimental.pallas import tpu_sc as plsc

_NUM_STOCKS = 100000
_EMBED_DIM = 32
_BATCH = 16384
_SEQ_LEN = 50

_B = _BATCH * _SEQ_LEN
_NC = 2
_NS = 16
_NW = _NC * _NS
_B_PER_W = _B // _NW
_IDXW = 128
_K = 20
_CHUNK = _K * _IDXW
_N_CHUNKS = _B_PER_W // _CHUNK

_mesh = plsc.VectorSubcoreMesh(core_axis_name="c", subcore_axis_name="s")


@functools.partial(
    pl.kernel,
    mesh=_mesh,
    out_type=jax.ShapeDtypeStruct((_B, _EMBED_DIM), jnp.float32),
    scratch_types=[
        pltpu.VMEM((_CHUNK,), jnp.int32),
        pltpu.VMEM((_CHUNK, _EMBED_DIM), jnp.float32),
        pltpu.SemaphoreType.DMA,
    ],
    compiler_params=pltpu.CompilerParams(use_tc_tiling_on_sc=False),
)
def _emb_lookup(idx_hbm, table_hbm, out_hbm, idx_v, rows_v, sem):
    wid = lax.axis_index("s") * _NC + lax.axis_index("c")
    base = wid * _B_PER_W

    def chunk_body(i, carry):
        off = base + i * _CHUNK
        pltpu.sync_copy(idx_hbm.at[pl.ds(off, _CHUNK)], idx_v)
        copies = []
        for j in range(_K):
            copies.append(
                pltpu.async_copy(
                    table_hbm.at[idx_v.at[pl.ds(j * _IDXW, _IDXW)]],
                    rows_v.at[pl.ds(j * _IDXW, _IDXW)],
                    sem,
                )
            )
        for cp in copies:
            cp.wait()
        pltpu.sync_copy(rows_v, out_hbm.at[pl.ds(off, _CHUNK)])
        return carry

    lax.fori_loop(0, _N_CHUNKS, chunk_body, 0)


def kernel(stock_ids, weight):
    idx_flat = stock_ids.reshape(_B)
    out = _emb_lookup(idx_flat, weight)
    return out.reshape(_BATCH, _SEQ_LEN, _EMBED_DIM)


# speedup vs baseline: 2.9976x; 1.0004x over previous
import functools

import jax
import jax.numpy as jnp
from jax import lax
from jax.experimental import pallas as pl
from jax.experimental.pallas import tpu as pltpu
from jax.experimental.pallas import tpu_sc as plsc

_NUM_STOCKS = 100000
_EMBED_DIM = 32
_BATCH = 16384
_SEQ_LEN = 50

_B = _BATCH * _SEQ_LEN
_NC = 2
_NS = 16
_NW = _NC * _NS
_B_PER_W = _B // _NW
_IDXW = 128
_K = 10
_CHUNK = _K * _IDXW
_N_CHUNKS = _B_PER_W // _CHUNK
_NBUF = 2

_mesh = plsc.VectorSubcoreMesh(core_axis_name="c", subcore_axis_name="s")


@functools.partial(
    pl.kernel,
    mesh=_mesh,
    out_type=jax.ShapeDtypeStruct((_B, _EMBED_DIM), jnp.float32),
    scratch_types=[
        pltpu.VMEM((_NBUF, _CHUNK), jnp.int32),
        pltpu.VMEM((_NBUF, _CHUNK, _EMBED_DIM), jnp.float32),
        pltpu.SemaphoreType.DMA((_NBUF,)),
        pltpu.SemaphoreType.DMA((_NBUF,)),
        pltpu.SemaphoreType.DMA((_NBUF,)),
    ],
    compiler_params=pltpu.CompilerParams(use_tc_tiling_on_sc=False),
)
def _emb_lookup(idx_hbm, table_hbm, out_hbm, idx_v, rows_v, isem, gsem, osem):
    wid = lax.axis_index("s") * _NC + lax.axis_index("c")
    base = wid * _B_PER_W

    for b in range(_NBUF):
        pltpu.async_copy(
            idx_hbm.at[pl.ds(base + b * _CHUNK, _CHUNK)], idx_v.at[b], isem.at[b]
        )

    def outer(it, carry):
        for b in range(_NBUF):
            i = it * _NBUF + b
            off = base + i * _CHUNK
            pltpu.make_async_copy(
                idx_hbm.at[pl.ds(base, _CHUNK)], idx_v.at[b], isem.at[b]
            ).wait()

            @pl.when(i >= _NBUF)
            def _():
                pltpu.make_async_copy(
                    rows_v.at[b], out_hbm.at[pl.ds(base, _CHUNK)], osem.at[b]
                ).wait()

            copies = []
            for j in range(_K):
                copies.append(
                    pltpu.async_copy(
                        table_hbm.at[idx_v.at[b].at[pl.ds(j * _IDXW, _IDXW)]],
                        rows_v.at[b].at[pl.ds(j * _IDXW, _IDXW)],
                        gsem.at[b],
                    )
                )
            for cp in copies:
                cp.wait()

            @pl.when(i + _NBUF < _N_CHUNKS)
            def _():
                pltpu.async_copy(
                    idx_hbm.at[pl.ds(off + _NBUF * _CHUNK, _CHUNK)],
                    idx_v.at[b],
                    isem.at[b],
                )

            pltpu.async_copy(rows_v.at[b], out_hbm.at[pl.ds(off, _CHUNK)], osem.at[b])
        return carry

    lax.fori_loop(0, _N_CHUNKS // _NBUF, outer, 0)

    for b in range(_NBUF):
        pltpu.make_async_copy(
            rows_v.at[b], out_hbm.at[pl.ds(base, _CHUNK)], osem.at[b]
        ).wait()


def kernel(stock_ids, weight):
    idx_flat = stock_ids.reshape(_B)
    out = _emb_lookup(idx_flat, weight)
    return out.reshape(_BATCH, _SEQ_LEN, _EMBED_DIM)


# speedup vs baseline: 4.6059x; 1.5365x over previous
import functools

import jax
import jax.numpy as jnp
from jax import lax
from jax.experimental import pallas as pl
from jax.experimental.pallas import tpu as pltpu
from jax.experimental.pallas import tpu_sc as plsc

_NUM_STOCKS = 100000
_EMBED_DIM = 32
_BATCH = 16384
_SEQ_LEN = 50

_B = _BATCH * _SEQ_LEN
_NC = 2
_NS = 16
_NW = _NC * _NS
_BW = _BATCH // _NW
_IDXW = 128
_NSTREAM = _BW // _IDXW

_mesh = plsc.VectorSubcoreMesh(core_axis_name="c", subcore_axis_name="s")


@functools.partial(
    pl.kernel,
    mesh=_mesh,
    out_type=jax.ShapeDtypeStruct((_SEQ_LEN, _EMBED_DIM, _BATCH), jnp.float32),
    scratch_types=[
        pltpu.VMEM((_BW * _SEQ_LEN,), jnp.int32),
        pltpu.VMEM((2, _BW), jnp.int32),
        pltpu.VMEM((2, _BW, _EMBED_DIM), jnp.float32),
        pltpu.VMEM((2, _EMBED_DIM, _BW), jnp.float32),
        pltpu.SemaphoreType.DMA,
        pltpu.SemaphoreType.DMA((2,)),
        pltpu.SemaphoreType.DMA((2,)),
    ],
    compiler_params=pltpu.CompilerParams(
        use_tc_tiling_on_sc=False, needs_layout_passes=False
    ),
)
def _emb_lookup(idx_hbm, table_hbm, out_hbm, idx_v, il_v, rows_v, tr_v,
                isem, gsem, osem):
    wid = lax.axis_index("s") * _NC + lax.axis_index("c")
    b0 = wid * _BW
    pltpu.async_copy(
        idx_hbm.at[pl.ds(b0 * _SEQ_LEN, _BW * _SEQ_LEN)], idx_v, isem
    ).wait()

    iota16 = lax.iota(jnp.int32, 16)
    iota_s = iota16 * _SEQ_LEN

    def build_ilist(p, s):
        for bb in range(_BW // 16):
            addr = iota_s + (bb * 16 * _SEQ_LEN + s)
            il_v[p, pl.ds(bb * 16, 16)] = plsc.load_gather(idx_v, [addr])

    def fire_gathers(p):
        for j in range(_NSTREAM):
            pltpu.async_copy(
                table_hbm.at[il_v.at[p].at[pl.ds(j * _IDXW, _IDXW)]],
                rows_v.at[p].at[pl.ds(j * _IDXW, _IDXW)],
                gsem.at[p],
            )

    def drain_gathers(p):
        for j in range(_NSTREAM):
            pltpu.make_async_copy(
                table_hbm.at[il_v.at[p].at[pl.ds(j * _IDXW, _IDXW)]],
                rows_v.at[p].at[pl.ds(j * _IDXW, _IDXW)],
                gsem.at[p],
            ).wait()

    def transpose(p):
        def dbody(d, carry):
            col = jnp.full((16,), d, jnp.int32)
            for bb in range(_BW // 16):
                rowi = iota16 + bb * 16
                tr_v[p, d, pl.ds(bb * 16, 16)] = plsc.load_gather(
                    rows_v.at[p], [rowi, col]
                )
            return carry
        lax.fori_loop(0, _EMBED_DIM, dbody, 0)

    build_ilist(0, 0)
    fire_gathers(0)

    def outer(it, carry):
        for p in range(2):
            s = it * 2 + p
            drain_gathers(p)

            @pl.when(s + 1 < _SEQ_LEN)
            def _():
                build_ilist(1 - p, s + 1)
                fire_gathers(1 - p)

            @pl.when(s >= 2)
            def _():
                pltpu.make_async_copy(
                    tr_v.at[p],
                    out_hbm.at[0].at[:, pl.ds(b0, _BW)],
                    osem.at[p],
                ).wait()

            transpose(p)
            pltpu.async_copy(
                tr_v.at[p], out_hbm.at[s].at[:, pl.ds(b0, _BW)], osem.at[p]
            )
        return carry

    lax.fori_loop(0, _SEQ_LEN // 2, outer, 0)

    for p in range(2):
        pltpu.make_async_copy(
            tr_v.at[p], out_hbm.at[0].at[:, pl.ds(b0, _BW)], osem.at[p]
        ).wait()


def kernel(stock_ids, weight):
    idx_flat = stock_ids.reshape(_B)
    out3 = _emb_lookup(idx_flat, weight)
    return out3.transpose(2, 0, 1)


# speedup vs baseline: 7.1766x; 1.5581x over previous
import functools

import jax
import jax.numpy as jnp
from jax import lax
from jax.experimental import pallas as pl
from jax.experimental.pallas import tpu as pltpu
from jax.experimental.pallas import tpu_sc as plsc

_NUM_STOCKS = 100000
_EMBED_DIM = 32
_BATCH = 16384
_SEQ_LEN = 50

_B = _BATCH * _SEQ_LEN
_NC = 2
_NS = 16
_NW = _NC * _NS
_BW = _BATCH // _NW
_IDXW = 128
_NSTREAM = _BW // _IDXW

_mesh = plsc.VectorSubcoreMesh(core_axis_name="c", subcore_axis_name="s")


@functools.partial(
    pl.kernel,
    mesh=_mesh,
    out_type=jax.ShapeDtypeStruct((_SEQ_LEN, _EMBED_DIM, _BATCH), jnp.float32),
    scratch_types=[
        pltpu.VMEM((_BW * _SEQ_LEN,), jnp.int32),
        pltpu.VMEM((2, _BW), jnp.int32),
        pltpu.VMEM((2, _BW, _EMBED_DIM), jnp.float32),
        pltpu.VMEM((2, _EMBED_DIM, _BW), jnp.float32),
        pltpu.SemaphoreType.DMA,
        pltpu.SemaphoreType.DMA((2,)),
        pltpu.SemaphoreType.DMA((2,)),
    ],
    compiler_params=pltpu.CompilerParams(
        use_tc_tiling_on_sc=False, needs_layout_passes=False
    ),
)
def _emb_lookup(idx_hbm, table_hbm, out_hbm, idx_v, il_v, rows_v, tr_v,
                isem, gsem, osem):
    wid = lax.axis_index("s") * _NC + lax.axis_index("c")
    b0 = wid * _BW
    pltpu.async_copy(
        idx_hbm.at[pl.ds(b0 * _SEQ_LEN, _BW * _SEQ_LEN)], idx_v, isem
    ).wait()

    iota16 = lax.iota(jnp.int32, 16)
    iota_s = iota16 * _SEQ_LEN

    def build_ilist(p, s):
        for bb in range(_BW // 16):
            addr = iota_s + (bb * 16 * _SEQ_LEN + s)
            il_v[p, pl.ds(bb * 16, 16)] = plsc.load_gather(idx_v, [addr])

    def fire_gathers(p):
        for j in range(_NSTREAM):
            pltpu.async_copy(
                table_hbm.at[il_v.at[p].at[pl.ds(j * _IDXW, _IDXW)]],
                rows_v.at[p].at[pl.ds(j * _IDXW, _IDXW)],
                gsem.at[p],
            )

    def drain_gathers(p):
        for j in range(_NSTREAM):
            pltpu.make_async_copy(
                table_hbm.at[il_v.at[p].at[pl.ds(j * _IDXW, _IDXW)]],
                rows_v.at[p].at[pl.ds(j * _IDXW, _IDXW)],
                gsem.at[p],
            ).wait()

    def transpose(p):
        @plsc.parallel_loop(0, _EMBED_DIM, step=1, unroll=4)
        def dbody(d):
            col = jnp.full((16,), d, jnp.int32)
            for bb in range(_BW // 16):
                rowi = iota16 + bb * 16
                tr_v[p, d, pl.ds(bb * 16, 16)] = plsc.load_gather(
                    rows_v.at[p], [rowi, col]
                )

    build_ilist(0, 0)
    fire_gathers(0)

    def outer(it, carry):
        for p in range(2):
            s = it * 2 + p
            drain_gathers(p)

            @pl.when(s + 1 < _SEQ_LEN)
            def _():
                build_ilist(1 - p, s + 1)
                fire_gathers(1 - p)

            @pl.when(s >= 2)
            def _():
                pltpu.make_async_copy(
                    tr_v.at[p],
                    out_hbm.at[0].at[:, pl.ds(b0, _BW)],
                    osem.at[p],
                ).wait()

            transpose(p)
            pltpu.async_copy(
                tr_v.at[p], out_hbm.at[s].at[:, pl.ds(b0, _BW)], osem.at[p]
            )
        return carry

    lax.fori_loop(0, _SEQ_LEN // 2, outer, 0)

    for p in range(2):
        pltpu.make_async_copy(
            tr_v.at[p], out_hbm.at[0].at[:, pl.ds(b0, _BW)], osem.at[p]
        ).wait()


def kernel(stock_ids, weight):
    idx_flat = stock_ids.reshape(_B)
    out3 = _emb_lookup(idx_flat, weight)
    return out3.transpose(2, 0, 1)
